# Initial kernel scaffold; baseline (speedup 1.0000x reference)
import jax
import jax.numpy as jnp
from jax.experimental import pallas as pl


def kernel(features, edge_index, edge_weight, W, Omega_1, V_W):
    raise NotImplementedError("write your pallas kernel here")


# speedup vs baseline: 11.4052x; 11.4052x over previous
import functools

import jax
import jax.numpy as jnp
from jax import lax
from jax.experimental import pallas as pl
from jax.experimental.pallas import tpu as pltpu
from jax.experimental.pallas import tpu_sc as plsc

_NHID = 128
_NCLASS = 64
_NNODE = 10000
_NEDGE = 160000
_KAPPA = 0.9
_FW_ITERS = 30
_RAD_ITERS = 100

_L = 16
_NS = 16
_NC = 2
_EPAD = 1280 * 128
_EROWS = _EPAD // 128
_NPAD = 10240
_SLICE = _NPAD // _NS
_ERPT = _EROWS // _NS
_ERPW = _EROWS // (_NC * _NS)
_RPT = _NNODE // _NS

_F32 = jnp.float32
_I32 = jnp.int32


def _power_body(row_hbm, col_hbm, w_hbm, out_hbm,
                er, ec, ew, v_loc, acc, pbuf, vsl, ssb1, ssb2, rb1, rb2,
                sh_p, sh_v, sh_s1, sh_s2):
    s = lax.axis_index("s")

    pltpu.sync_copy(row_hbm.at[pl.ds(s * _ERPT, _ERPT)], er)
    pltpu.sync_copy(col_hbm.at[pl.ds(s * _ERPT, _ERPT)], ec)
    pltpu.sync_copy(w_hbm.at[pl.ds(s * _ERPT, _ERPT)], ew)

    def _abs_row(r, _):
        for k in range(8):
            ew[r, pl.ds(k * _L, _L)] = jnp.abs(ew[r, pl.ds(k * _L, _L)])
        return 0
    lax.fori_loop(0, _ERPT, _abs_row, 0)

    v0 = jnp.full((_L,), 1.0 / _NNODE, _F32)
    zz = jnp.zeros((_L,), _F32)

    def _init_v(g, _):
        v_loc[pl.ds(g * _L, _L)] = v0
        return 0
    lax.fori_loop(0, _NNODE // _L, _init_v, 0)

    def _init_pad(g, _):
        v_loc[pl.ds(g * _L, _L)] = zz
        return 0
    lax.fori_loop(_NNODE // _L, _NPAD // _L, _init_pad, 0)

    def _zero_acc(g, _):
        acc[pl.ds(g * _L, _L)] = zz
        return 0

    def _edges(r, _):
        for k in range(8):
            sl = pl.ds(k * _L, _L)
            c16 = ec[r, sl]
            g16 = plsc.load_gather(v_loc, [c16])
            w16 = ew[r, sl]
            r16 = er[r, sl]
            plsc.addupdate_scatter(acc, [r16], g16 * w16)
        return 0

    def _matvec_to_slice(want_sq):
        lax.fori_loop(0, _NPAD // _L, _zero_acc, 0)
        lax.fori_loop(0, _ERPT, _edges, 0)
        pltpu.sync_copy(acc, sh_p.at[s])
        plsc.subcore_barrier()
        for t in range(_NS):
            pltpu.sync_copy(sh_p.at[t, pl.ds(s * _SLICE, _SLICE)], pbuf.at[t])

        def _red(g, ssum):
            sl = pl.ds(g * _L, _L)
            tot = pbuf[0, sl]
            for t in range(1, _NS):
                tot = tot + pbuf[t, sl]
            vsl[sl] = tot
            if want_sq:
                return ssum + tot * tot
            return ssum + tot
        return lax.fori_loop(0, _SLICE // _L, _red, jnp.zeros((_L,), _F32))

    def _iter(it, _):
        ssum = _matvec_to_slice(False)
        tot_scalar = jnp.sum(ssum)
        rb1[...] = jnp.zeros((_L,), _F32) + tot_scalar
        pltpu.sync_copy(vsl, sh_v.at[pl.ds(s * _SLICE, _SLICE)])
        pltpu.sync_copy(rb1, sh_s1.at[s])
        plsc.subcore_barrier()
        pltpu.sync_copy(sh_v, v_loc)
        pltpu.sync_copy(sh_s1, ssb1)
        tot16 = ssb1[0]
        for t in range(1, _NS):
            tot16 = tot16 + ssb1[t]
        inv16 = 1.0 / tot16

        def _scale(g, _):
            sl = pl.ds(g * _L, _L)
            v_loc[sl] = v_loc[sl] * inv16
            return 0
        lax.fori_loop(0, _NPAD // _L, _scale, 0)
        return 0

    lax.fori_loop(0, _RAD_ITERS - 1, _iter, 0)

    def _sq_v(g, ssum):
        x = v_loc[pl.ds(s * _SLICE + g * _L, _L)]
        return ssum + x * x
    sq_d = lax.fori_loop(0, _SLICE // _L, _sq_v, jnp.zeros((_L,), _F32))
    sq_n = _matvec_to_slice(True)
    rb1[...] = jnp.zeros((_L,), _F32) + jnp.sum(sq_n)
    rb2[...] = jnp.zeros((_L,), _F32) + jnp.sum(sq_d)
    pltpu.sync_copy(rb1, sh_s1.at[s])
    pltpu.sync_copy(rb2, sh_s2.at[s])
    plsc.subcore_barrier()
    pltpu.sync_copy(sh_s1, ssb1)
    pltpu.sync_copy(sh_s2, ssb2)
    sn = ssb1[0]
    sd = ssb2[0]
    for t in range(1, _NS):
        sn = sn + ssb1[t]
        sd = sd + ssb2[t]
    x = sn / sd
    xi = plsc.bitcast(x, _I32)
    yi = jnp.full((_L,), 0x5F3759DF, _I32) - lax.shift_right_logical(
        xi, jnp.full((_L,), 1, _I32))
    y = plsc.bitcast(yi, _F32)
    half_x = 0.5 * x
    for _ in range(3):
        y = y * (1.5 - half_x * y * y)
    rho = x * y + 1e-5

    @pl.when(s == 0)
    def _():
        rb1[...] = rho
        pltpu.sync_copy(rb1, out_hbm)


_SC_PARAMS = pltpu.CompilerParams(
    needs_layout_passes=False, use_tc_tiling_on_sc=False)

_sc_power = functools.partial(
    pl.kernel,
    out_type=jax.ShapeDtypeStruct((_L,), _F32),
    compiler_params=_SC_PARAMS,
    mesh=plsc.VectorSubcoreMesh(
        core_axis_name="c", subcore_axis_name="s", num_cores=1),
    scratch_types=[
        pltpu.VMEM((_ERPT, 128), _I32),
        pltpu.VMEM((_ERPT, 128), _I32),
        pltpu.VMEM((_ERPT, 128), _F32),
        pltpu.VMEM((_NPAD,), _F32),
        pltpu.VMEM((_NPAD,), _F32),
        pltpu.VMEM((_NS, _SLICE), _F32),
        pltpu.VMEM((_SLICE,), _F32),
        pltpu.VMEM((_NS, _L), _F32),
        pltpu.VMEM((_NS, _L), _F32),
        pltpu.VMEM((_L,), _F32),
        pltpu.VMEM((_L,), _F32),
        pltpu.VMEM_SHARED((_NS, _NPAD), _F32),
        pltpu.VMEM_SHARED((_NPAD,), _F32),
        pltpu.VMEM_SHARED((_NS, _L), _F32),
        pltpu.VMEM_SHARED((_NS, _L), _F32),
    ],
)(_power_body)


def _spmm_body(y_hbm, row_hbm, col_hbm, w_hbm, out_hbm,
               rid, cid, wv, rows, acc_sh, sem):
    c = lax.axis_index("c")
    s = lax.axis_index("s")
    wid = c * _NS + s

    zz = jnp.zeros((_L,), _F32)

    def _zero_rows(e, _):
        for k in range(8):
            rows[e, pl.ds(k * _L, _L)] = zz
        return 0
    lax.fori_loop(0, 128, _zero_rows, 0)

    for j in range(4):
        pltpu.sync_copy(rows, acc_sh.at[pl.ds(s * _RPT + j * 128, 128)])
    pltpu.sync_copy(rows.at[pl.ds(0, _RPT - 512)],
                    acc_sh.at[pl.ds(s * _RPT + 512, _RPT - 512)])
    plsc.subcore_barrier()

    def _chunk(j, _):
        base = wid * _ERPW + j
        pltpu.sync_copy(row_hbm.at[base], rid)
        pltpu.sync_copy(col_hbm.at[base], cid)
        pltpu.sync_copy(w_hbm.at[base], wv)
        pltpu.async_copy(y_hbm.at[rid], rows, sem).wait()

        def _scale(e, _):
            ws = plsc.load_gather(wv, [lax.broadcast(e, (_L,))])
            for k in range(8):
                sl = pl.ds(k * _L, _L)
                rows[e, sl] = rows[e, sl] * ws
            return 0
        lax.fori_loop(0, 128, _scale, 0)
        pltpu.sync_copy(rows, acc_sh.at[cid], add=True)
        return 0
    lax.fori_loop(0, _ERPW, _chunk, 0)

    plsc.subcore_barrier()
    for j in range(4):
        pltpu.sync_copy(acc_sh.at[pl.ds(s * _RPT + j * 128, 128)],
                        out_hbm.at[c, pl.ds(s * _RPT + j * 128, 128)])
    pltpu.sync_copy(acc_sh.at[pl.ds(s * _RPT + 512, _RPT - 512)],
                    out_hbm.at[c, pl.ds(s * _RPT + 512, _RPT - 512)])


_sc_spmm = functools.partial(
    pl.kernel,
    out_type=jax.ShapeDtypeStruct((_NC, _NNODE, _NHID), _F32),
    compiler_params=_SC_PARAMS,
    mesh=plsc.VectorSubcoreMesh(
        core_axis_name="c", subcore_axis_name="s", num_cores=_NC),
    scratch_types=[
        pltpu.VMEM((128,), _I32),
        pltpu.VMEM((128,), _I32),
        pltpu.VMEM((128,), _F32),
        pltpu.VMEM((128, _NHID), _F32),
        pltpu.VMEM_SHARED((_NNODE, _NHID), _F32),
        pltpu.SemaphoreType.DMA,
    ],
)(_spmm_body)


_BR = 1000


def _fuse_body(sp_ref, bp_ref, wp_ref, y_ref):
    x = sp_ref[0] + sp_ref[1] + bp_ref[0] + bp_ref[1]
    x = jnp.maximum(x, 0.0)
    y_ref[...] = lax.dot_general(
        x, wp_ref[...], (((1,), (1,)), ((), ())),
        precision=lax.Precision.HIGHEST, preferred_element_type=_F32)


_tc_fuse = pl.pallas_call(
    _fuse_body,
    grid=(_NNODE // _BR,),
    in_specs=[
        pl.BlockSpec((_NC, _BR, _NHID), lambda i: (0, i, 0)),
        pl.BlockSpec((_NC, _BR, _NHID), lambda i: (0, i, 0)),
        pl.BlockSpec((_NHID, _NHID), lambda i: (0, 0)),
    ],
    out_specs=pl.BlockSpec((_BR, _NHID), lambda i: (i, 0)),
    out_shape=jax.ShapeDtypeStruct((_NNODE, _NHID), _F32),
)


def _final_body(sp_ref, bp_ref, vw_ref, out_ref):
    x = sp_ref[0] + sp_ref[1] + bp_ref[0] + bp_ref[1]
    x = jnp.maximum(x, 0.0)
    nrm = jnp.sqrt(jnp.sum(x * x, axis=1, keepdims=True))
    x = x / jnp.maximum(nrm, 1e-12)
    out_ref[...] = lax.dot_general(
        x, vw_ref[...], (((1,), (1,)), ((), ())),
        precision=lax.Precision.HIGHEST, preferred_element_type=_F32)


_tc_final = pl.pallas_call(
    _final_body,
    grid=(_NNODE // _BR,),
    in_specs=[
        pl.BlockSpec((_NC, _BR, _NHID), lambda i: (0, i, 0)),
        pl.BlockSpec((_NC, _BR, _NHID), lambda i: (0, i, 0)),
        pl.BlockSpec((_NHID, _NHID), lambda i: (0, 0)),
    ],
    out_specs=pl.BlockSpec((_BR, _NHID), lambda i: (i, 0)),
    out_shape=jax.ShapeDtypeStruct((_NNODE, _NHID), _F32),
)


def _s1t_body(feat_ref, om_ref, out_ref):
    out_ref[...] = lax.dot_general(
        feat_ref[...], om_ref[...], (((0,), (1,)), ((), ())),
        precision=lax.Precision.HIGHEST, preferred_element_type=_F32)


_BC = 1280

_tc_s1t = pl.pallas_call(
    _s1t_body,
    grid=(_NPAD // _BC,),
    in_specs=[
        pl.BlockSpec((_NHID, _BC), lambda i: (0, i)),
        pl.BlockSpec((_NHID, _NHID), lambda i: (0, 0)),
    ],
    out_specs=pl.BlockSpec((_BC, _NHID), lambda i: (i, 0)),
    out_shape=jax.ShapeDtypeStruct((_NPAD, _NHID), _F32),
)


def _proj_body(w_ref, vv_ref, wp_ref):
    w = w_ref[...]
    vv = vv_ref[0, 0]
    a = jnp.abs(w)
    srow = jnp.sum(a, axis=1, keepdims=True)
    lo = jnp.zeros((_NHID, 1), _F32)
    hi = srow

    def _bis(i, carry):
        lo, hi = carry
        mid = 0.5 * (lo + hi)
        ssum = jnp.sum(jnp.maximum(a - mid, 0.0), axis=1, keepdims=True)
        pred = ssum > vv
        return (jnp.where(pred, mid, lo), jnp.where(pred, hi, mid))

    lo, hi = lax.fori_loop(0, 50, _bis, (lo, hi))
    theta = 0.5 * (lo + hi)
    wp = jnp.sign(w) * jnp.maximum(a - theta, 0.0)
    wp_ref[...] = jnp.where(srow > vv, wp, w)


_tc_project = pl.pallas_call(
    _proj_body,
    grid=(1,),
    in_specs=[
        pl.BlockSpec((_NHID, _NHID), lambda i: (0, 0)),
        pl.BlockSpec((8, 128), lambda i: (0, 0)),
    ],
    out_specs=pl.BlockSpec((_NHID, _NHID), lambda i: (0, 0)),
    out_shape=jax.ShapeDtypeStruct((_NHID, _NHID), _F32),
)


def kernel(features, edge_index, edge_weight, W, Omega_1, V_W):
    row = edge_index[0].astype(_I32)
    col = edge_index[1].astype(_I32)
    w = edge_weight.astype(_F32)
    pad = _EPAD - _NEDGE
    zi = jnp.zeros((pad,), _I32)
    row_p = jnp.concatenate([row, zi]).reshape(_EROWS, 128)
    col_p = jnp.concatenate([col, zi]).reshape(_EROWS, 128)
    w_p = jnp.concatenate([w, jnp.zeros((pad,), _F32)]).reshape(_EROWS, 128)

    rho16 = _sc_power(row_p, col_p, w_p)
    vv_arr = jnp.full((8, 128), _KAPPA / rho16[0], _F32)
    Wp = _tc_project(W, vv_arr)

    feat_pad = jnp.pad(features, ((0, 0), (0, _NPAD - _NNODE)))
    s1t = _tc_s1t(feat_pad, Omega_1)[: _NNODE]
    b_parts = _sc_spmm(s1t, row_p, col_p, w_p)

    zeros_parts = jnp.zeros((_NC, _NNODE, _NHID), _F32)
    y = _tc_fuse(zeros_parts, b_parts, Wp)

    def _body(i, y):
        return _tc_fuse(_sc_spmm(y, row_p, col_p, w_p), b_parts, Wp)

    y = lax.fori_loop(0, _FW_ITERS - 2, _body, y)

    vw_pad = jnp.concatenate(
        [V_W.astype(_F32), jnp.zeros((_NHID - _NCLASS, _NHID), _F32)], axis=0)
    out = _tc_final(_sc_spmm(y, row_p, col_p, w_p), b_parts, vw_pad)
    return out[:, :_NCLASS]


# speedup vs baseline: 16.9623x; 1.4872x over previous
import functools

import jax
import jax.numpy as jnp
from jax import lax
from jax.experimental import pallas as pl
from jax.experimental.pallas import tpu as pltpu
from jax.experimental.pallas import tpu_sc as plsc

_NHID = 128
_NCLASS = 64
_NNODE = 10000
_NEDGE = 160000
_KAPPA = 0.9
_FW_ITERS = 30
_RAD_ITERS = 100

_L = 16
_NS = 16
_NC = 2
_EPAD = 1280 * 128
_EROWS = _EPAD // 128
_NPAD = 10240
_SLICE = _NPAD // _NS
_ERPT = _EROWS // _NS
_ERPW = _EROWS // (_NC * _NS)
_RPT = _NNODE // _NS

_F32 = jnp.float32
_I32 = jnp.int32


def _power_body(row_hbm, col_hbm, w_hbm, out_hbm,
                er, ec, ew, v_loc, acc, pbuf, vsl, ssb1, ssb2, rb1, rb2,
                sh_p, sh_v, sh_s1, sh_s2, psem):
    s = lax.axis_index("s")

    pltpu.sync_copy(row_hbm.at[pl.ds(s * _ERPT, _ERPT)], er)
    pltpu.sync_copy(col_hbm.at[pl.ds(s * _ERPT, _ERPT)], ec)
    pltpu.sync_copy(w_hbm.at[pl.ds(s * _ERPT, _ERPT)], ew)

    def _abs_row(r, _):
        for k in range(8):
            ew[r, pl.ds(k * _L, _L)] = jnp.abs(ew[r, pl.ds(k * _L, _L)])
        return 0
    lax.fori_loop(0, _ERPT, _abs_row, 0)

    v0 = jnp.full((_L,), 1.0 / _NNODE, _F32)
    zz = jnp.zeros((_L,), _F32)

    def _init_v(g, _):
        v_loc[pl.ds(g * _L, _L)] = v0
        return 0
    lax.fori_loop(0, _NNODE // _L, _init_v, 0)

    def _init_pad(g, _):
        v_loc[pl.ds(g * _L, _L)] = zz
        return 0
    lax.fori_loop(_NNODE // _L, _NPAD // _L, _init_pad, 0)

    def _zero_acc(g, _):
        for k in range(8):
            acc[pl.ds((g * 8 + k) * _L, _L)] = zz
        return 0

    def _edges(r, _):
        for k in range(8):
            sl = pl.ds(k * _L, _L)
            c16 = ec[r, sl]
            g16 = plsc.load_gather(v_loc, [c16])
            w16 = ew[r, sl]
            r16 = er[r, sl]
            plsc.addupdate_scatter(acc, [r16], g16 * w16)
        return 0

    def _matvec_to_slice(want_sq):
        lax.fori_loop(0, _NPAD // (_L * 8), _zero_acc, 0)
        lax.fori_loop(0, _ERPT, _edges, 0)
        pltpu.sync_copy(acc, sh_p.at[s])
        plsc.subcore_barrier()
        for t in range(_NS):
            pltpu.async_copy(
                sh_p.at[t, pl.ds(s * _SLICE, _SLICE)], pbuf.at[t], psem)
        for t in range(_NS):
            pltpu.make_async_copy(
                sh_p.at[t, pl.ds(s * _SLICE, _SLICE)], pbuf.at[t], psem).wait()

        def _red(g, ssum):
            sl = pl.ds(g * _L, _L)
            tot = pbuf[0, sl]
            for t in range(1, _NS):
                tot = tot + pbuf[t, sl]
            vsl[sl] = tot
            if want_sq:
                return ssum + tot * tot
            return ssum + tot
        return lax.fori_loop(0, _SLICE // _L, _red, jnp.zeros((_L,), _F32))

    def _iter(it, _):
        ssum = _matvec_to_slice(False)
        tot_scalar = jnp.sum(ssum)
        rb1[...] = jnp.zeros((_L,), _F32) + tot_scalar
        pltpu.sync_copy(vsl, sh_v.at[pl.ds(s * _SLICE, _SLICE)])
        pltpu.sync_copy(rb1, sh_s1.at[s])
        plsc.subcore_barrier()
        pltpu.sync_copy(sh_v, v_loc)
        pltpu.sync_copy(sh_s1, ssb1)
        tot16 = ssb1[0]
        for t in range(1, _NS):
            tot16 = tot16 + ssb1[t]
        inv16 = 1.0 / tot16

        def _scale(g, _):
            for k in range(8):
                sl = pl.ds((g * 8 + k) * _L, _L)
                v_loc[sl] = v_loc[sl] * inv16
            return 0
        lax.fori_loop(0, _NPAD // (_L * 8), _scale, 0)
        return 0

    lax.fori_loop(0, _RAD_ITERS - 1, _iter, 0)

    def _sq_v(g, ssum):
        x = v_loc[pl.ds(s * _SLICE + g * _L, _L)]
        return ssum + x * x
    sq_d = lax.fori_loop(0, _SLICE // _L, _sq_v, jnp.zeros((_L,), _F32))
    sq_n = _matvec_to_slice(True)
    rb1[...] = jnp.zeros((_L,), _F32) + jnp.sum(sq_n)
    rb2[...] = jnp.zeros((_L,), _F32) + jnp.sum(sq_d)
    pltpu.sync_copy(rb1, sh_s1.at[s])
    pltpu.sync_copy(rb2, sh_s2.at[s])
    plsc.subcore_barrier()
    pltpu.sync_copy(sh_s1, ssb1)
    pltpu.sync_copy(sh_s2, ssb2)
    sn = ssb1[0]
    sd = ssb2[0]
    for t in range(1, _NS):
        sn = sn + ssb1[t]
        sd = sd + ssb2[t]
    x = sn / sd
    xi = plsc.bitcast(x, _I32)
    yi = jnp.full((_L,), 0x5F3759DF, _I32) - lax.shift_right_logical(
        xi, jnp.full((_L,), 1, _I32))
    y = plsc.bitcast(yi, _F32)
    half_x = 0.5 * x
    for _ in range(3):
        y = y * (1.5 - half_x * y * y)
    rho = x * y + 1e-5

    @pl.when(s == 0)
    def _():
        rb1[...] = rho
        pltpu.sync_copy(rb1, out_hbm)


_SC_PARAMS = pltpu.CompilerParams(
    needs_layout_passes=False, use_tc_tiling_on_sc=False)

_sc_power = functools.partial(
    pl.kernel,
    out_type=jax.ShapeDtypeStruct((_L,), _F32),
    compiler_params=_SC_PARAMS,
    mesh=plsc.VectorSubcoreMesh(
        core_axis_name="c", subcore_axis_name="s", num_cores=1),
    scratch_types=[
        pltpu.VMEM((_ERPT, 128), _I32),
        pltpu.VMEM((_ERPT, 128), _I32),
        pltpu.VMEM((_ERPT, 128), _F32),
        pltpu.VMEM((_NPAD,), _F32),
        pltpu.VMEM((_NPAD,), _F32),
        pltpu.VMEM((_NS, _SLICE), _F32),
        pltpu.VMEM((_SLICE,), _F32),
        pltpu.VMEM((_NS, _L), _F32),
        pltpu.VMEM((_NS, _L), _F32),
        pltpu.VMEM((_L,), _F32),
        pltpu.VMEM((_L,), _F32),
        pltpu.VMEM_SHARED((_NS, _NPAD), _F32),
        pltpu.VMEM_SHARED((_NPAD,), _F32),
        pltpu.VMEM_SHARED((_NS, _L), _F32),
        pltpu.VMEM_SHARED((_NS, _L), _F32),
        pltpu.SemaphoreType.DMA,
    ],
)(_power_body)


def _spmm_body(y_hbm, row_hbm, col_hbm, w_hbm, out_hbm,
               ridA, cidA, wvA, rows0, rows1, acc_sh, esem, gsem):
    c = lax.axis_index("c")
    s = lax.axis_index("s")
    wid = c * _NS + s
    base = wid * _ERPW

    pltpu.async_copy(row_hbm.at[pl.ds(base, _ERPW + 1)], ridA, esem)
    pltpu.async_copy(col_hbm.at[pl.ds(base, _ERPW)], cidA, esem)
    pltpu.async_copy(w_hbm.at[pl.ds(base, _ERPW)], wvA, esem)

    zz = jnp.zeros((_L,), _F32)

    def _zero_rows(e, _):
        for k in range(8):
            rows0[e, pl.ds(k * _L, _L)] = zz
        return 0
    lax.fori_loop(0, 128, _zero_rows, 0)

    pltpu.make_async_copy(row_hbm.at[pl.ds(base, _ERPW + 1)], ridA, esem).wait()
    pltpu.make_async_copy(col_hbm.at[pl.ds(base, _ERPW)], cidA, esem).wait()
    pltpu.make_async_copy(w_hbm.at[pl.ds(base, _ERPW)], wvA, esem).wait()

    for j in range(4):
        pltpu.sync_copy(rows0, acc_sh.at[pl.ds(s * _RPT + j * 128, 128)])
    pltpu.sync_copy(rows0.at[pl.ds(0, _RPT - 512)],
                    acc_sh.at[pl.ds(s * _RPT + 512, _RPT - 512)])
    plsc.subcore_barrier()

    def _gather(j, buf):
        pltpu.async_copy(y_hbm.at[ridA.at[j]], buf, gsem)

    def _process(j, buf):
        pltpu.make_async_copy(y_hbm.at[ridA.at[j]], buf, gsem).wait()

        def _grp(g, _):
            for e in range(_L):
                ws = plsc.load_gather(
                    wvA, [lax.broadcast(j, (_L,)),
                          lax.broadcast(g * _L + e, (_L,))])
                r = g * _L + e
                for k in range(8):
                    sl = pl.ds(k * _L, _L)
                    buf[r, sl] = buf[r, sl] * ws
            return 0
        lax.fori_loop(0, 128 // _L, _grp, 0)
        pltpu.sync_copy(buf, acc_sh.at[cidA.at[j]], add=True)

    _gather(0, rows0)

    def _pair(j2, _):
        j = 2 * j2
        _gather(j + 1, rows1)
        _process(j, rows0)
        _gather(j + 2, rows0)
        _process(j + 1, rows1)
        return 0
    lax.fori_loop(0, _ERPW // 2, _pair, 0)
    pltpu.make_async_copy(y_hbm.at[ridA.at[0]], rows0, gsem).wait()

    plsc.subcore_barrier()
    for j in range(4):
        pltpu.sync_copy(acc_sh.at[pl.ds(s * _RPT + j * 128, 128)],
                        out_hbm.at[c, pl.ds(s * _RPT + j * 128, 128)])
    pltpu.sync_copy(acc_sh.at[pl.ds(s * _RPT + 512, _RPT - 512)],
                    out_hbm.at[c, pl.ds(s * _RPT + 512, _RPT - 512)])


_sc_spmm = functools.partial(
    pl.kernel,
    out_type=jax.ShapeDtypeStruct((_NC, _NNODE, _NHID), _F32),
    compiler_params=_SC_PARAMS,
    mesh=plsc.VectorSubcoreMesh(
        core_axis_name="c", subcore_axis_name="s", num_cores=_NC),
    scratch_types=[
        pltpu.VMEM((_ERPW + 1, 128), _I32),
        pltpu.VMEM((_ERPW, 128), _I32),
        pltpu.VMEM((_ERPW, 128), _F32),
        pltpu.VMEM((128, _NHID), _F32),
        pltpu.VMEM((128, _NHID), _F32),
        pltpu.VMEM_SHARED((_NNODE, _NHID), _F32),
        pltpu.SemaphoreType.DMA,
        pltpu.SemaphoreType.DMA,
    ],
)(_spmm_body)


_BR = 1000


def _fuse_body(sp_ref, bp_ref, wp_ref, y_ref):
    x = sp_ref[0] + sp_ref[1] + bp_ref[0] + bp_ref[1]
    x = jnp.maximum(x, 0.0)
    y_ref[...] = lax.dot_general(
        x, wp_ref[...], (((1,), (1,)), ((), ())),
        precision=lax.Precision.HIGHEST, preferred_element_type=_F32)


_tc_fuse = pl.pallas_call(
    _fuse_body,
    grid=(_NNODE // _BR,),
    in_specs=[
        pl.BlockSpec((_NC, _BR, _NHID), lambda i: (0, i, 0)),
        pl.BlockSpec((_NC, _BR, _NHID), lambda i: (0, i, 0)),
        pl.BlockSpec((_NHID, _NHID), lambda i: (0, 0)),
    ],
    out_specs=pl.BlockSpec((_BR, _NHID), lambda i: (i, 0)),
    out_shape=jax.ShapeDtypeStruct((_NNODE, _NHID), _F32),
)


def _final_body(sp_ref, bp_ref, vw_ref, out_ref):
    x = sp_ref[0] + sp_ref[1] + bp_ref[0] + bp_ref[1]
    x = jnp.maximum(x, 0.0)
    nrm = jnp.sqrt(jnp.sum(x * x, axis=1, keepdims=True))
    x = x / jnp.maximum(nrm, 1e-12)
    out_ref[...] = lax.dot_general(
        x, vw_ref[...], (((1,), (1,)), ((), ())),
        precision=lax.Precision.HIGHEST, preferred_element_type=_F32)


_tc_final = pl.pallas_call(
    _final_body,
    grid=(_NNODE // _BR,),
    in_specs=[
        pl.BlockSpec((_NC, _BR, _NHID), lambda i: (0, i, 0)),
        pl.BlockSpec((_NC, _BR, _NHID), lambda i: (0, i, 0)),
        pl.BlockSpec((_NHID, _NHID), lambda i: (0, 0)),
    ],
    out_specs=pl.BlockSpec((_BR, _NHID), lambda i: (i, 0)),
    out_shape=jax.ShapeDtypeStruct((_NNODE, _NHID), _F32),
)


def _s1t_body(feat_ref, om_ref, out_ref):
    out_ref[...] = lax.dot_general(
        feat_ref[...], om_ref[...], (((0,), (1,)), ((), ())),
        precision=lax.Precision.HIGHEST, preferred_element_type=_F32)


_BC = 1280

_tc_s1t = pl.pallas_call(
    _s1t_body,
    grid=(_NPAD // _BC,),
    in_specs=[
        pl.BlockSpec((_NHID, _BC), lambda i: (0, i)),
        pl.BlockSpec((_NHID, _NHID), lambda i: (0, 0)),
    ],
    out_specs=pl.BlockSpec((_BC, _NHID), lambda i: (i, 0)),
    out_shape=jax.ShapeDtypeStruct((_NPAD, _NHID), _F32),
)


def _proj_body(w_ref, vv_ref, wp_ref):
    w = w_ref[...]
    vv = vv_ref[0, 0]
    a = jnp.abs(w)
    srow = jnp.sum(a, axis=1, keepdims=True)
    lo = jnp.zeros((_NHID, 1), _F32)
    hi = srow

    def _bis(i, carry):
        lo, hi = carry
        mid = 0.5 * (lo + hi)
        ssum = jnp.sum(jnp.maximum(a - mid, 0.0), axis=1, keepdims=True)
        pred = ssum > vv
        return (jnp.where(pred, mid, lo), jnp.where(pred, hi, mid))

    lo, hi = lax.fori_loop(0, 50, _bis, (lo, hi))
    theta = 0.5 * (lo + hi)
    wp = jnp.sign(w) * jnp.maximum(a - theta, 0.0)
    wp_ref[...] = jnp.where(srow > vv, wp, w)


_tc_project = pl.pallas_call(
    _proj_body,
    grid=(1,),
    in_specs=[
        pl.BlockSpec((_NHID, _NHID), lambda i: (0, 0)),
        pl.BlockSpec((8, 128), lambda i: (0, 0)),
    ],
    out_specs=pl.BlockSpec((_NHID, _NHID), lambda i: (0, 0)),
    out_shape=jax.ShapeDtypeStruct((_NHID, _NHID), _F32),
)


def kernel(features, edge_index, edge_weight, W, Omega_1, V_W):
    row = edge_index[0].astype(_I32)
    col = edge_index[1].astype(_I32)
    w = edge_weight.astype(_F32)
    pad = _EPAD - _NEDGE + 128
    zi = jnp.zeros((pad,), _I32)
    row_p = jnp.concatenate([row, zi]).reshape(_EROWS + 1, 128)
    col_p = jnp.concatenate([col, zi]).reshape(_EROWS + 1, 128)
    w_p = jnp.concatenate(
        [w, jnp.zeros((pad,), _F32)]).reshape(_EROWS + 1, 128)

    rho16 = _sc_power(row_p, col_p, w_p)
    vv_arr = jnp.full((8, 128), _KAPPA / rho16[0], _F32)
    Wp = _tc_project(W, vv_arr)

    feat_pad = jnp.pad(features, ((0, 0), (0, _NPAD - _NNODE)))
    s1t = _tc_s1t(feat_pad, Omega_1)[: _NNODE]
    b_parts = _sc_spmm(s1t, row_p, col_p, w_p)

    zeros_parts = jnp.zeros((_NC, _NNODE, _NHID), _F32)
    y = _tc_fuse(zeros_parts, b_parts, Wp)

    def _body(i, y):
        return _tc_fuse(_sc_spmm(y, row_p, col_p, w_p), b_parts, Wp)

    y = lax.fori_loop(0, _FW_ITERS - 2, _body, y)

    vw_pad = jnp.concatenate(
        [V_W.astype(_F32), jnp.zeros((_NHID - _NCLASS, _NHID), _F32)], axis=0)
    out = _tc_final(_sc_spmm(y, row_p, col_p, w_p), b_parts, vw_pad)
    return out[:, :_NCLASS]
